# Initial kernel scaffold; baseline (speedup 1.0000x reference)
import jax
import jax.numpy as jnp
from jax.experimental import pallas as pl


def kernel(q_sub, q_rel, hidden, edges, n_node, rela_embed, Ws, Wr, Wqr_w, Wqr_b, wa_w, wa_b, W_h):
    raise NotImplementedError("write your pallas kernel here")


# speedup vs baseline: 1.4717x; 1.4717x over previous
import functools

import jax
import jax.numpy as jnp
from jax import lax
from jax.experimental import pallas as pl
from jax.experimental.pallas import tpu as pltpu
from jax.experimental.pallas import tpu_sc as plsc

F32 = jnp.float32
I32 = jnp.int32

N_NODE = 10000
E_EDGE = 160000
DIM = 256
HALF = 128
REL_PAD = 408

E_PAD = 161280
EPT1 = E_PAD // 32
NB1 = 80
NBLK1 = EPT1 // NB1

EPT2 = E_EDGE // 16
NB2 = 80
NBLK2 = EPT2 // NB2
STRIPE = 624
TAIL = N_NODE - 16 * STRIPE


def _prep_hidden_body(x_ref, w_ref, a_ref, h2_ref):
    x = x_ref[...]
    a_ref[...] = jnp.dot(x, w_ref[...], preferred_element_type=F32)
    h2_ref[0] = x[:, :HALF]
    h2_ref[1] = x[:, HALF:]


def _prep_rela_body(x_ref, wr_ref, wq_ref, b_ref, rw_ref, qa_ref, r2_ref):
    x = x_ref[...]
    rw_ref[...] = jnp.dot(x, wr_ref[...], preferred_element_type=F32)
    qa_ref[...] = jnp.dot(x, wq_ref[...], preferred_element_type=F32) + b_ref[...]
    r2_ref[0] = x[:, :HALF]
    r2_ref[1] = x[:, HALF:]


def _final_body(g_ref, w_ref, o_ref):
    y = jnp.dot(g_ref[0], w_ref[0], preferred_element_type=F32)
    y = y + jnp.dot(g_ref[1], w_ref[1], preferred_element_type=F32)
    o_ref[...] = jnp.maximum(y, 0.0)


_SC_MESH = plsc.VectorSubcoreMesh(core_axis_name="c", subcore_axis_name="s")


@functools.partial(
    pl.kernel,
    mesh=_SC_MESH,
    compiler_params=pltpu.CompilerParams(needs_layout_passes=False),
    out_type=jax.ShapeDtypeStruct((E_PAD,), F32),
    scratch_types=[
        pltpu.VMEM((NB1,), I32),
        pltpu.VMEM((NB1,), I32),
        pltpu.VMEM((NB1,), I32),
        pltpu.VMEM((NB1, DIM), F32),
        pltpu.VMEM((NB1, DIM), F32),
        pltpu.VMEM((64, DIM), F32),
        pltpu.VMEM((64,), I32),
        pltpu.VMEM((DIM,), F32),
        pltpu.VMEM((16,), F32),
        pltpu.VMEM((NB1,), F32),
        pltpu.SemaphoreType.DMA,
    ],
)
def _alpha_kernel(a_hbm, rw_hbm, qa_hbm, sub_hbm, rel_hbm, ridx_hbm, qrel_hbm,
                  wa_hbm, wab_hbm, alpha_hbm,
                  sub_v, rel_v, ridx_v, rows_a, rows_r, rows_qb,
                  qrel_v, wa_v, wab_v, alpha_v, sem):
    cid = lax.axis_index("c")
    sid = lax.axis_index("s")
    base = (sid * 2 + cid) * EPT1
    pltpu.sync_copy(qrel_hbm, qrel_v)
    pltpu.sync_copy(wa_hbm, wa_v)
    pltpu.sync_copy(wab_hbm, wab_v)
    pltpu.async_copy(qa_hbm.at[qrel_v], rows_qb, sem).wait()

    def block(b, carry):
        off = base + b * NB1
        pltpu.sync_copy(sub_hbm.at[pl.ds(off, NB1)], sub_v)
        pltpu.sync_copy(rel_hbm.at[pl.ds(off, NB1)], rel_v)
        pltpu.sync_copy(ridx_hbm.at[pl.ds(off, NB1)], ridx_v)
        ca = pltpu.async_copy(a_hbm.at[sub_v], rows_a, sem)
        cr = pltpu.async_copy(rw_hbm.at[rel_v], rows_r, sem)
        ca.wait()
        cr.wait()
        wab = wab_v[...]
        lanes = lax.broadcasted_iota(I32, (16,), 0)

        def group(g, carry2):
            s16 = jnp.zeros((16,), F32)
            r16 = ridx_v[pl.ds(g * 16, 16)]
            for k in range(16):
                e = g * 16 + k
                rq = r16[k]
                acc = jnp.zeros((16,), F32)
                for c in range(16):
                    sl = pl.ds(c * 16, 16)
                    v = rows_a[e, sl] + rows_r[e, sl] + rows_qb[rq, sl]
                    acc = acc + jnp.maximum(v, 0.0) * wa_v[sl]
                s16 = jnp.where(lanes == k, jnp.sum(acc), s16)
            s16 = s16 + wab
            alpha_v[pl.ds(g * 16, 16)] = 1.0 / (1.0 + jnp.exp(-s16))
            return carry2

        lax.fori_loop(0, NB1 // 16, group, 0)
        pltpu.sync_copy(alpha_v, alpha_hbm.at[pl.ds(off, NB1)])
        return carry

    lax.fori_loop(0, NBLK1, block, 0)


@functools.partial(
    pl.kernel,
    mesh=_SC_MESH,
    compiler_params=pltpu.CompilerParams(needs_layout_passes=False),
    out_type=jax.ShapeDtypeStruct((2 * N_NODE, HALF), F32),
    scratch_types=[
        pltpu.VMEM((NB2,), I32),
        pltpu.VMEM((NB2,), I32),
        pltpu.VMEM((NB2,), I32),
        pltpu.VMEM((NB2,), I32),
        pltpu.VMEM((NB2,), I32),
        pltpu.VMEM((NB2,), F32),
        pltpu.VMEM((NB2, HALF), F32),
        pltpu.VMEM((NB2, HALF), F32),
        pltpu.VMEM((NB2, HALF), F32),
        pltpu.VMEM_SHARED((N_NODE, HALF), F32),
        pltpu.SemaphoreType.DMA,
    ],
)
def _scatter_kernel(h2_hbm, r2_hbm, alpha_hbm, sub_hbm, rel_hbm, obj_hbm,
                    z_hbm, out_hbm,
                    sub_v, rel_v, obj_v, hidx_v, ridx_v, alpha_v,
                    rows_h, rows_r, msg_v, acc_sp, sem):
    cid = lax.axis_index("c")
    sid = lax.axis_index("s")
    base = sid * EPT2
    pltpu.sync_copy(z_hbm, acc_sp.at[pl.ds(pl.multiple_of(sid * STRIPE, 8), STRIPE)])

    @pl.when(sid == 15)
    def _zero_tail():
        pltpu.sync_copy(z_hbm.at[pl.ds(0, TAIL)],
                        acc_sp.at[pl.ds(16 * STRIPE, TAIL)])

    plsc.subcore_barrier()
    hoff = cid * N_NODE
    roff = cid * REL_PAD

    def block(b, carry):
        off = base + b * NB2
        pltpu.sync_copy(sub_hbm.at[pl.ds(off, NB2)], sub_v)
        pltpu.sync_copy(rel_hbm.at[pl.ds(off, NB2)], rel_v)
        pltpu.sync_copy(obj_hbm.at[pl.ds(off, NB2)], obj_v)
        pltpu.sync_copy(alpha_hbm.at[pl.ds(off, NB2)], alpha_v)
        for j in range(NB2 // 16):
            sl = pl.ds(j * 16, 16)
            hidx_v[sl] = sub_v[sl] + hoff
            ridx_v[sl] = rel_v[sl] + roff
        ch = pltpu.async_copy(h2_hbm.at[hidx_v], rows_h, sem)
        cr = pltpu.async_copy(r2_hbm.at[ridx_v], rows_r, sem)
        ch.wait()
        cr.wait()

        def group(g, carry2):
            a16 = alpha_v[pl.ds(g * 16, 16)]
            for k in range(16):
                e = g * 16 + k
                a = a16[k]
                for c in range(HALF // 16):
                    sl = pl.ds(c * 16, 16)
                    msg_v[e, sl] = (rows_h[e, sl] + rows_r[e, sl]) * a
            return carry2

        lax.fori_loop(0, NB2 // 16, group, 0)
        pltpu.sync_copy(msg_v, acc_sp.at[obj_v], add=True)
        return carry

    lax.fori_loop(0, NBLK2, block, 0)
    plsc.subcore_barrier()
    pltpu.sync_copy(
        acc_sp.at[pl.ds(pl.multiple_of(sid * STRIPE, 8), STRIPE)],
        out_hbm.at[pl.ds(pl.multiple_of(cid * N_NODE + sid * STRIPE, 8), STRIPE)])

    @pl.when(sid == 15)
    def _out_tail():
        pltpu.sync_copy(
            acc_sp.at[pl.ds(16 * STRIPE, TAIL)],
            out_hbm.at[pl.ds(pl.multiple_of(cid * N_NODE + 16 * STRIPE, 8), TAIL)])


def kernel(q_sub, q_rel, hidden, edges, n_node, rela_embed, Ws, Wr, Wqr_w,
           Wqr_b, wa_w, wa_b, W_h):
    del q_sub, n_node
    edges = edges.astype(I32)
    sub = edges[:, 4]
    rel = edges[:, 2]
    obj = edges[:, 5]
    ridx = edges[:, 0]
    pad = E_PAD - E_EDGE
    sub_p = jnp.pad(sub, (0, pad))
    rel_p = jnp.pad(rel, (0, pad))
    ridx_p = jnp.pad(ridx, (0, pad))

    blk = 1000
    nblk = N_NODE // blk
    A, h2 = pl.pallas_call(
        _prep_hidden_body,
        grid=(nblk,),
        in_specs=[pl.BlockSpec((blk, DIM), lambda i: (i, 0)),
                  pl.BlockSpec((DIM, DIM), lambda i: (0, 0))],
        out_specs=[pl.BlockSpec((blk, DIM), lambda i: (i, 0)),
                   pl.BlockSpec((2, blk, HALF), lambda i: (0, i, 0))],
        out_shape=[jax.ShapeDtypeStruct((N_NODE, DIM), F32),
                   jax.ShapeDtypeStruct((2, N_NODE, HALF), F32)],
    )(hidden, Ws)

    rela_pad = jnp.pad(rela_embed, ((0, REL_PAD - rela_embed.shape[0]), (0, 0)))
    bq = Wqr_b.reshape(1, DIM)
    RW, QA, r2 = pl.pallas_call(
        _prep_rela_body,
        grid=(1,),
        in_specs=[pl.BlockSpec((REL_PAD, DIM), lambda i: (0, 0)),
                  pl.BlockSpec((DIM, DIM), lambda i: (0, 0)),
                  pl.BlockSpec((DIM, DIM), lambda i: (0, 0)),
                  pl.BlockSpec((1, DIM), lambda i: (0, 0))],
        out_specs=[pl.BlockSpec((REL_PAD, DIM), lambda i: (0, 0)),
                   pl.BlockSpec((REL_PAD, DIM), lambda i: (0, 0)),
                   pl.BlockSpec((2, REL_PAD, HALF), lambda i: (0, 0, 0))],
        out_shape=[jax.ShapeDtypeStruct((REL_PAD, DIM), F32),
                   jax.ShapeDtypeStruct((REL_PAD, DIM), F32),
                   jax.ShapeDtypeStruct((2, REL_PAD, HALF), F32)],
    )(rela_pad, Wr, Wqr_w, bq)

    qrel_i = q_rel.astype(I32)
    wa_vec = wa_w.reshape(DIM).astype(F32)
    wab_vec = jnp.full((16,), wa_b.reshape(()), dtype=F32)

    alpha = _alpha_kernel(A, RW, QA, sub_p, rel_p, ridx_p, qrel_i, wa_vec,
                          wab_vec)

    zrows = jnp.zeros((STRIPE, HALF), F32)
    agg = _scatter_kernel(h2.reshape(2 * N_NODE, HALF),
                          r2.reshape(2 * REL_PAD, HALF),
                          alpha, sub, rel, obj, zrows)

    out = pl.pallas_call(
        _final_body,
        grid=(nblk,),
        in_specs=[pl.BlockSpec((2, blk, HALF), lambda i: (0, i, 0)),
                  pl.BlockSpec((2, HALF, DIM), lambda i: (0, 0, 0))],
        out_specs=pl.BlockSpec((blk, DIM), lambda i: (i, 0)),
        out_shape=jax.ShapeDtypeStruct((N_NODE, DIM), F32),
    )(agg.reshape(2, N_NODE, HALF), W_h.reshape(2, HALF, DIM))
    return out
